# baseline (speedup 1.0000x reference)
import functools

import jax
import jax.numpy as jnp
from jax import lax
from jax.experimental import pallas as pl
from jax.experimental.pallas import tpu as pltpu
from jax.experimental.pallas import tpu_sc as plsc

DIM = 32
SEQ = 50
BATCH = 4096
N = BATCH * SEQ
KDIM = SEQ * DIM
H1 = 10
H2 = 5

_info = plsc.get_sparse_core_info()
NC, NS = _info.num_cores, _info.num_subcores
NW = NC * NS
BW_B = BATCH // NW
BW = BW_B * SEQ
MEGA_B = 64
NMEGA = BW_B // MEGA_B
MEGA_R = MEGA_B * SEQ
NGRP = MEGA_B // 16
JB = 5


def _splat(x):
    return jnp.broadcast_to(x, (16,))


def _fused(idx_flat, emb, w1f, b1p, w2f, b2p):
    mesh = plsc.VectorSubcoreMesh(core_axis_name="c", subcore_axis_name="s")

    @functools.partial(
        pl.kernel,
        mesh=mesh,
        out_type=jax.ShapeDtypeStruct((BATCH, H2), jnp.float32),
        compiler_params=pltpu.CompilerParams(
            use_tc_tiling_on_sc=False, needs_layout_passes=False
        ),
        scratch_types=[
            pltpu.VMEM((BW,), jnp.int32),
            pltpu.VMEM((MEGA_R, DIM), jnp.float32),
            pltpu.VMEM((KDIM * H1,), jnp.float32),
            pltpu.VMEM((16,), jnp.float32),
            pltpu.VMEM((64,), jnp.float32),
            pltpu.VMEM((16,), jnp.float32),
            pltpu.VMEM((BW_B, H1), jnp.float32),
            pltpu.VMEM((BW_B, H2), jnp.float32),
            pltpu.SemaphoreType.DMA,
        ],
    )
    def k(idx_hbm, table_hbm, w1_hbm, b1_hbm, w2_hbm, b2_hbm, out_hbm,
          idx_v, rows_v, w1_v, b1_v, w2_v, b2_v, h_v, out_v, sem):
        wid = lax.axis_index("s") * NC + lax.axis_index("c")
        base = wid * BW
        pltpu.sync_copy(idx_hbm.at[pl.ds(base, BW)], idx_v)
        pltpu.sync_copy(w1_hbm, w1_v)
        pltpu.sync_copy(b1_hbm, b1_v)
        pltpu.sync_copy(w2_hbm, w2_v)
        pltpu.sync_copy(b2_hbm, b2_v)

        lane = lax.iota(jnp.int32, 16)
        zidx = lane * 0

        for m in range(NMEGA):
            pltpu.async_copy(
                table_hbm.at[idx_v.at[pl.ds(m * MEGA_R, MEGA_R)]],
                rows_v, sem,
            ).wait()
            for jb in range(H1 // JB):
                def body(kk, accs):
                    s = kk // DIM
                    d = kk - s * DIM
                    col = _splat(d)
                    xs = [
                        plsc.load_gather(
                            rows_v,
                            [lane * SEQ + _splat(g * 16 * SEQ + s), col],
                        )
                        for g in range(NGRP)
                    ]
                    out = []
                    for jj in range(JB):
                        w = plsc.load_gather(
                            w1_v, [_splat(kk * H1 + jb * JB + jj)]
                        )
                        for g in range(NGRP):
                            out.append(accs[jj * NGRP + g] + xs[g] * w)
                    return tuple(out)

                accs = lax.fori_loop(
                    0, KDIM, body,
                    tuple(jnp.zeros((16,), jnp.float32)
                          for _ in range(JB * NGRP)),
                )
                for jj in range(JB):
                    for g in range(NGRP):
                        brow = lane + _splat(m * MEGA_B + g * 16)
                        plsc.store_scatter(
                            h_v, [brow, _splat(jb * JB + jj)],
                            accs[jj * NGRP + g],
                        )

        b1s = [plsc.load_gather(b1_v, [_splat(j)]) for j in range(H1)]
        b2s = [plsc.load_gather(b2_v, [_splat(o)]) for o in range(H2)]
        w2s = [[plsc.load_gather(w2_v, [_splat(j * H2 + o)])
                for o in range(H2)] for j in range(H1)]
        for g8 in range(BW_B // 16):
            brow = lane + _splat(g8 * 16)
            hs = [
                jnp.maximum(
                    plsc.load_gather(h_v, [brow, _splat(j)]) + b1s[j], 0.0
                )
                for j in range(H1)
            ]
            for o in range(H2):
                acc = b2s[o]
                for j in range(H1):
                    acc = acc + hs[j] * w2s[j][o]
                sig = 1.0 / (1.0 + jnp.exp(-acc))
                plsc.store_scatter(out_v, [brow, _splat(o)], sig)
        pltpu.sync_copy(out_v, out_hbm.at[pl.ds(wid * BW_B, BW_B)])

    return k(idx_flat, emb, w1f, b1p, w2f, b2p)


def kernel(indices, emb, W1, b1, W2, b2):
    idx_flat = indices.astype(jnp.int32).reshape(N)
    w1f = W1.reshape(KDIM * H1)
    b1p = jnp.pad(b1, (0, 16 - H1))
    w2f = jnp.pad(W2.reshape(H1 * H2), (0, 64 - H1 * H2))
    b2p = jnp.pad(b2, (0, 16 - H2))
    return _fused(idx_flat, emb, w1f, b1p, w2f, b2p)


# speedup vs baseline: 1.0015x; 1.0015x over previous
import functools

import jax
import jax.numpy as jnp
from jax import lax
from jax.experimental import pallas as pl
from jax.experimental.pallas import tpu as pltpu
from jax.experimental.pallas import tpu_sc as plsc

DIM = 32
SEQ = 50
BATCH = 4096
N = BATCH * SEQ
KDIM = SEQ * DIM
H1 = 10
H2 = 5

_info = plsc.get_sparse_core_info()
NC, NS = _info.num_cores, _info.num_subcores
NW = NC * NS
BW_B = BATCH // NW
BW = BW_B * SEQ
MEGA_B = 64
NMEGA = BW_B // MEGA_B
MEGA_R = MEGA_B * SEQ
NGRP = MEGA_B // 16
JB = 5


def _splat(x):
    return jnp.broadcast_to(x, (16,))


def _fused(idx_flat, emb, w1f, b1p, w2f, b2p):
    mesh = plsc.VectorSubcoreMesh(core_axis_name="c", subcore_axis_name="s")

    @functools.partial(
        pl.kernel,
        mesh=mesh,
        out_type=jax.ShapeDtypeStruct((BATCH, H2), jnp.float32),
        compiler_params=pltpu.CompilerParams(
            use_tc_tiling_on_sc=False, needs_layout_passes=False
        ),
        scratch_types=[
            pltpu.VMEM((BW_B, SEQ), jnp.int32),
            pltpu.VMEM((MEGA_R, DIM), jnp.float32),
            pltpu.VMEM((KDIM * H1,), jnp.float32),
            pltpu.VMEM((16,), jnp.float32),
            pltpu.VMEM((64,), jnp.float32),
            pltpu.VMEM((16,), jnp.float32),
            pltpu.VMEM((BW_B, H1), jnp.float32),
            pltpu.VMEM((BW_B, H2), jnp.float32),
            pltpu.SemaphoreType.DMA,
        ],
    )
    def k(idx_hbm, table_hbm, w1_hbm, b1_hbm, w2_hbm, b2_hbm, out_hbm,
          idx_v, rows_v, w1_v, b1_v, w2_v, b2_v, h_v, out_v, sem):
        wid = lax.axis_index("s") * NC + lax.axis_index("c")
        pltpu.sync_copy(idx_hbm.at[pl.ds(wid * BW_B, BW_B), :], idx_v)
        pltpu.sync_copy(w1_hbm, w1_v)
        pltpu.sync_copy(b1_hbm, b1_v)
        pltpu.sync_copy(w2_hbm, w2_v)
        pltpu.sync_copy(b2_hbm, b2_v)

        lane = lax.iota(jnp.int32, 16)

        for m in range(NMEGA):
            def fire(r, carry):
                pltpu.async_copy(
                    table_hbm.at[idx_v.at[m * MEGA_B + r]],
                    rows_v.at[pl.ds(r * SEQ, SEQ)],
                    sem,
                )
                return carry

            lax.fori_loop(0, MEGA_B, fire, 0)
            pltpu.make_async_copy(
                table_hbm.at[pl.ds(0, MEGA_R)], rows_v, sem
            ).wait()

            for jb in range(H1 // JB):
                def body(s, accs):
                    rvecs = [
                        lane * SEQ + _splat(g * 16 * SEQ + s)
                        for g in range(NGRP)
                    ]
                    wbase = s * (DIM * H1) + jb * JB
                    out = list(accs)
                    for d in range(32):
                        col = _splat(jnp.int32(d))
                        xs = [
                            plsc.load_gather(rows_v, [rvecs[g], col])
                            for g in range(NGRP)
                        ]
                        for jj in range(JB):
                            w = plsc.load_gather(
                                w1_v, [_splat(wbase + (d * H1 + jj))]
                            )
                            for g in range(NGRP):
                                i = jj * NGRP + g
                                out[i] = out[i] + xs[g] * w
                    return tuple(out)

                accs = lax.fori_loop(
                    0, SEQ, body,
                    tuple(jnp.zeros((16,), jnp.float32)
                          for _ in range(JB * NGRP)),
                )
                for jj in range(JB):
                    for g in range(NGRP):
                        brow = lane + _splat(m * MEGA_B + g * 16)
                        plsc.store_scatter(
                            h_v, [brow, _splat(jb * JB + jj)],
                            accs[jj * NGRP + g],
                        )

        b1s = [plsc.load_gather(b1_v, [_splat(j)]) for j in range(H1)]
        b2s = [plsc.load_gather(b2_v, [_splat(o)]) for o in range(H2)]
        w2s = [[plsc.load_gather(w2_v, [_splat(j * H2 + o)])
                for o in range(H2)] for j in range(H1)]
        for g8 in range(BW_B // 16):
            brow = lane + _splat(g8 * 16)
            hs = [
                jnp.maximum(
                    plsc.load_gather(h_v, [brow, _splat(j)]) + b1s[j], 0.0
                )
                for j in range(H1)
            ]
            for o in range(H2):
                acc = b2s[o]
                for j in range(H1):
                    acc = acc + hs[j] * w2s[j][o]
                sig = 1.0 / (1.0 + jnp.exp(-acc))
                plsc.store_scatter(out_v, [brow, _splat(o)], sig)
        pltpu.sync_copy(out_v, out_hbm.at[pl.ds(wid * BW_B, BW_B)])

    return k(idx_flat, emb, w1f, b1p, w2f, b2p)


def kernel(indices, emb, W1, b1, W2, b2):
    idx_flat = indices.astype(jnp.int32)
    w1f = W1.reshape(KDIM * H1)
    b1p = jnp.pad(b1, (0, 16 - H1))
    w2f = jnp.pad(W2.reshape(H1 * H2), (0, 64 - H1 * H2))
    b2p = jnp.pad(b2, (0, 16 - H2))
    return _fused(idx_flat, emb, w1f, b1p, w2f, b2p)


# speedup vs baseline: 1.4016x; 1.3994x over previous
import functools

import jax
import jax.numpy as jnp
from jax import lax
from jax.experimental import pallas as pl
from jax.experimental.pallas import tpu as pltpu
from jax.experimental.pallas import tpu_sc as plsc

DIM = 32
SEQ = 50
BATCH = 4096
KDIM = SEQ * DIM
H1 = 10
H2 = 5

_info = plsc.get_sparse_core_info()
NC, NS = _info.num_cores, _info.num_subcores
NW = NC * NS
BW_B = BATCH // NW
MEGA_B = 32
NMEGA = BW_B // MEGA_B
MEGA_R = MEGA_B * SEQ
BB = 4
JB = 5


def _splat(x):
    return jnp.broadcast_to(x, (16,))


def _fused(idx2d, emb, w1, b1p, w2f, b2p):
    mesh = plsc.VectorSubcoreMesh(core_axis_name="c", subcore_axis_name="s")

    @functools.partial(
        pl.kernel,
        mesh=mesh,
        out_type=jax.ShapeDtypeStruct((BATCH, H2), jnp.float32),
        compiler_params=pltpu.CompilerParams(
            use_tc_tiling_on_sc=False, needs_layout_passes=False
        ),
        scratch_types=[
            pltpu.VMEM((BW_B, SEQ), jnp.int32),
            pltpu.VMEM((MEGA_R, DIM), jnp.float32),
            pltpu.VMEM((KDIM, H1), jnp.float32),
            pltpu.VMEM((H1, KDIM), jnp.float32),
            pltpu.VMEM((16,), jnp.float32),
            pltpu.VMEM((64,), jnp.float32),
            pltpu.VMEM((16,), jnp.float32),
            pltpu.VMEM((BW_B, H1), jnp.float32),
            pltpu.VMEM((BW_B, H2), jnp.float32),
            pltpu.SemaphoreType.DMA,
        ],
    )
    def k(idx_hbm, table_hbm, w1_hbm, b1_hbm, w2_hbm, b2_hbm, out_hbm,
          idx_v, rows_v, w1_v, w1t_v, b1_v, w2_v, b2_v, h_v, out_v, sem):
        wid = lax.axis_index("s") * NC + lax.axis_index("c")
        pltpu.sync_copy(idx_hbm.at[pl.ds(wid * BW_B, BW_B), :], idx_v)
        pltpu.sync_copy(w1_hbm, w1_v)
        pltpu.sync_copy(b1_hbm, b1_v)
        pltpu.sync_copy(w2_hbm, w2_v)
        pltpu.sync_copy(b2_hbm, b2_v)

        lane = lax.iota(jnp.int32, 16)
        last = lane == _splat(jnp.int32(15))

        def wt_body(t, carry):
            base = _splat(t * 16) + lane
            for j in range(H1):
                v = plsc.load_gather(w1_v, [base, _splat(jnp.int32(j))])
                w1t_v[j, pl.ds(t * 16, 16)] = v
            return carry

        lax.fori_loop(0, KDIM // 16, wt_body, 0)

        for m in range(NMEGA):
            def fire(r, carry):
                pltpu.async_copy(
                    table_hbm.at[idx_v.at[m * MEGA_B + r]],
                    rows_v.at[pl.ds(r * SEQ, SEQ)],
                    sem,
                )
                return carry

            lax.fori_loop(0, MEGA_B, fire, 0)
            pltpu.make_async_copy(
                table_hbm.at[pl.ds(0, MEGA_R)], rows_v, sem
            ).wait()

            def bb_body(bb, carry):
                for jb in range(H1 // JB):
                    def body(s, accs):
                        out = list(accs)
                        xs = []
                        for bi in range(BB):
                            r = (bb * BB + bi) * SEQ + s
                            xs.append((rows_v[r, pl.ds(0, 16)],
                                       rows_v[r, pl.ds(16, 16)]))
                        for jj in range(JB):
                            j = jb * JB + jj
                            w0 = w1t_v[j, pl.ds(s * DIM, 16)]
                            w1r = w1t_v[j, pl.ds(s * DIM + 16, 16)]
                            for bi in range(BB):
                                i = jj * BB + bi
                                out[i] = (out[i] + xs[bi][0] * w0
                                          + xs[bi][1] * w1r)
                        return tuple(out)

                    accs = lax.fori_loop(
                        0, SEQ, body,
                        tuple(jnp.zeros((16,), jnp.float32)
                              for _ in range(JB * BB)),
                    )
                    for jj in range(JB):
                        for bi in range(BB):
                            cum = jnp.cumsum(accs[jj * BB + bi])
                            bg = m * MEGA_B + bb * BB + bi
                            plsc.store_scatter(
                                h_v, [_splat(bg), _splat(jnp.int32(jb * JB + jj))],
                                cum, mask=last,
                            )
                return carry

            lax.fori_loop(0, MEGA_B // BB, bb_body, 0)

        b1s = [plsc.load_gather(b1_v, [_splat(jnp.int32(j))])
               for j in range(H1)]
        b2s = [plsc.load_gather(b2_v, [_splat(jnp.int32(o))])
               for o in range(H2)]
        w2s = [[plsc.load_gather(w2_v, [_splat(jnp.int32(j * H2 + o))])
                for o in range(H2)] for j in range(H1)]

        def l2_body(g8, carry):
            brow = lane + _splat(g8 * 16)
            hs = [
                jnp.maximum(
                    plsc.load_gather(h_v, [brow, _splat(jnp.int32(j))])
                    + b1s[j], 0.0
                )
                for j in range(H1)
            ]
            for o in range(H2):
                acc = b2s[o]
                for j in range(H1):
                    acc = acc + hs[j] * w2s[j][o]
                sig = 1.0 / (1.0 + jnp.exp(-acc))
                plsc.store_scatter(out_v, [brow, _splat(jnp.int32(o))], sig)
            return carry

        lax.fori_loop(0, BW_B // 16, l2_body, 0)
        pltpu.sync_copy(out_v, out_hbm.at[pl.ds(wid * BW_B, BW_B)])

    return k(idx2d, emb, w1, b1p, w2f, b2p)


def kernel(indices, emb, W1, b1, W2, b2):
    idx2d = indices.astype(jnp.int32)
    b1p = jnp.pad(b1, (0, 16 - H1))
    w2f = jnp.pad(W2.reshape(H1 * H2), (0, 64 - H1 * H2))
    b2p = jnp.pad(b2, (0, 16 - H2))
    return _fused(idx2d, emb, W1, b1p, w2f, b2p)
